# Initial kernel scaffold; baseline (speedup 1.0000x reference)
import jax
import jax.numpy as jnp
from jax.experimental import pallas as pl


def kernel(x, rbf, idnb_i, n_atoms, W_rbf, W1, b1, W2, b2, W3, b3, W_out):
    raise NotImplementedError("write your pallas kernel here")


# speedup vs baseline: 2.9514x; 2.9514x over previous
import functools

import jax
import jax.numpy as jnp
from jax import lax
from jax.experimental import pallas as pl
from jax.experimental.pallas import tpu as pltpu
from jax.experimental.pallas import tpu_sc as plsc

E = 320000
N = 10000
D = 128
R = 16
T = 12

_EDGE_BLK = 2560


def _edge_body(x_ref, rbf_ref, w_ref, h_ref):
    g = jnp.dot(rbf_ref[...], w_ref[...], preferred_element_type=jnp.float32)
    h_ref[...] = g * x_ref[...]


def _edge_stage(x, rbf, W_rbf):
    grid = (E // _EDGE_BLK,)
    return pl.pallas_call(
        _edge_body,
        grid=grid,
        in_specs=[
            pl.BlockSpec((_EDGE_BLK, D), lambda i: (i, 0)),
            pl.BlockSpec((_EDGE_BLK, R), lambda i: (i, 0)),
            pl.BlockSpec((R, D), lambda i: (0, 0)),
        ],
        out_specs=pl.BlockSpec((_EDGE_BLK, D), lambda i: (i, 0)),
        out_shape=jax.ShapeDtypeStruct((E, D), jnp.float32),
    )(x, rbf, W_rbf)


_NC = 2
_NS = 16
_EW = E // (_NC * _NS)
_CH = 128
_CF = _EW // _CH
_TAIL = _EW - _CF * _CH
N_PAD = 10112
_RPW = N_PAD // _NS

_sc_mesh = plsc.VectorSubcoreMesh(core_axis_name="c", subcore_axis_name="s")


@functools.partial(
    pl.kernel,
    out_type=jax.ShapeDtypeStruct((_NC, N_PAD, D), jnp.float32),
    mesh=_sc_mesh,
    scratch_types=[
        pltpu.VMEM_SHARED((N_PAD, D), jnp.float32),
        pltpu.VMEM((_CH, D), jnp.float32),
        pltpu.VMEM((_CH, D), jnp.float32),
        pltpu.VMEM((_CH,), jnp.int32),
        pltpu.VMEM((_CH,), jnp.int32),
        pltpu.VMEM((_TAIL, D), jnp.float32),
        pltpu.VMEM((_TAIL,), jnp.int32),
        pltpu.SemaphoreType.DMA,
        pltpu.SemaphoreType.DMA,
        pltpu.SemaphoreType.DMA,
        pltpu.SemaphoreType.DMA,
    ],
)
def _scatter_stage(h_hbm, idx_hbm, zeros_hbm, out_hbm,
                   acc, b0, b1, i0, i1, bt, it, sr0, sr1, si0, si1):
    c = lax.axis_index("c")
    s = lax.axis_index("s")
    base = (c * _NS + s) * _EW

    pltpu.sync_copy(zeros_hbm.at[pl.ds(s * _RPW, _RPW)],
                    acc.at[pl.ds(s * _RPW, _RPW)])
    plsc.subcore_barrier()

    def start(chunk, rb, ib, sr, si):
        off = base + chunk * _CH
        pltpu.async_copy(h_hbm.at[pl.ds(off, _CH)], rb, sr)
        pltpu.async_copy(idx_hbm.at[pl.ds(off, _CH)], ib, si)

    def wait(rb, ib, sr, si):
        pltpu.make_async_copy(h_hbm.at[pl.ds(0, _CH)], rb, sr).wait()
        pltpu.make_async_copy(idx_hbm.at[pl.ds(0, _CH)], ib, si).wait()

    start(0, b0, i0, sr0, si0)
    start(1, b1, i1, sr1, si1)

    def body(j, carry):
        wait(b0, i0, sr0, si0)
        pltpu.sync_copy(b0, acc.at[i0], add=True)
        start(2 * j + 2, b0, i0, sr0, si0)
        wait(b1, i1, sr1, si1)
        pltpu.sync_copy(b1, acc.at[i1], add=True)
        start(2 * j + 3, b1, i1, sr1, si1)
        return carry

    lax.fori_loop(0, _CF // 2 - 1, body, 0)

    wait(b0, i0, sr0, si0)
    pltpu.sync_copy(b0, acc.at[i0], add=True)
    wait(b1, i1, sr1, si1)
    pltpu.sync_copy(b1, acc.at[i1], add=True)

    toff = base + _CF * _CH
    pltpu.sync_copy(h_hbm.at[pl.ds(toff, _TAIL)], bt)
    pltpu.sync_copy(idx_hbm.at[pl.ds(toff, _TAIL)], it)
    pltpu.sync_copy(bt, acc.at[it], add=True)

    plsc.subcore_barrier()
    pltpu.sync_copy(acc.at[pl.ds(s * _RPW, _RPW)],
                    out_hbm.at[c, pl.ds(s * _RPW, _RPW)])


_NODE_BLK = 1264


def _mlp_body(p_ref, w1_ref, b1_ref, w2_ref, b2_ref, w3_ref, b3_ref, wo_ref,
              out_ref):
    h = p_ref[0] + p_ref[1]
    h = jax.nn.silu(jnp.dot(h, w1_ref[...],
                            preferred_element_type=jnp.float32) + b1_ref[...])
    h = jax.nn.silu(jnp.dot(h, w2_ref[...],
                            preferred_element_type=jnp.float32) + b2_ref[...])
    h = jax.nn.silu(jnp.dot(h, w3_ref[...],
                            preferred_element_type=jnp.float32) + b3_ref[...])
    out_ref[...] = jnp.dot(h, wo_ref[...], preferred_element_type=jnp.float32)


def _mlp_stage(partials, W1, b1, W2, b2, W3, b3, W_out_pad):
    grid = (N_PAD // _NODE_BLK,)
    full = lambda i: (0, 0)
    return pl.pallas_call(
        _mlp_body,
        grid=grid,
        in_specs=[
            pl.BlockSpec((_NC, _NODE_BLK, D), lambda i: (0, i, 0)),
            pl.BlockSpec((D, D), full),
            pl.BlockSpec((1, D), full),
            pl.BlockSpec((D, D), full),
            pl.BlockSpec((1, D), full),
            pl.BlockSpec((D, D), full),
            pl.BlockSpec((1, D), full),
            pl.BlockSpec((D, D), full),
        ],
        out_specs=pl.BlockSpec((_NODE_BLK, D), lambda i: (i, 0)),
        out_shape=jax.ShapeDtypeStruct((N_PAD, D), jnp.float32),
    )(partials, W1, b1, W2, b2, W3, b3, W_out_pad)


def kernel(x, rbf, idnb_i, n_atoms, W_rbf, W1, b1, W2, b2, W3, b3, W_out):
    del n_atoms
    h = _edge_stage(x, rbf, W_rbf)
    zeros = jnp.zeros((N_PAD, D), dtype=jnp.float32)
    partials = _scatter_stage(h, idnb_i, zeros)
    W_out_pad = jnp.pad(W_out, ((0, 0), (0, D - T)))
    out = _mlp_stage(partials, W1, b1.reshape(1, D), W2, b2.reshape(1, D),
                     W3, b3.reshape(1, D), W_out_pad)
    return out[:N, :T]


# speedup vs baseline: 3.1215x; 1.0577x over previous
import functools

import jax
import jax.numpy as jnp
from jax import lax
from jax.experimental import pallas as pl
from jax.experimental.pallas import tpu as pltpu
from jax.experimental.pallas import tpu_sc as plsc

E = 320000
N = 10000
D = 128
R = 16
T = 12

_NSLICE = 2
_ES = E // _NSLICE

_EDGE_BLK = 3200
_BPS = _ES // _EDGE_BLK


def _edge_body(x_ref, rbf_ref, w_ref, h_ref):
    g = jnp.dot(rbf_ref[...], w_ref[...], preferred_element_type=jnp.float32)
    h_ref[...] = g * x_ref[...]


def _edge_stage(x, rbf, W_rbf, sl):
    off = sl * _BPS
    return pl.pallas_call(
        _edge_body,
        grid=(_BPS,),
        in_specs=[
            pl.BlockSpec((_EDGE_BLK, D), lambda i: (i + off, 0)),
            pl.BlockSpec((_EDGE_BLK, R), lambda i: (i + off, 0)),
            pl.BlockSpec((R, D), lambda i: (0, 0)),
        ],
        out_specs=pl.BlockSpec((_EDGE_BLK, D), lambda i: (i, 0)),
        out_shape=jax.ShapeDtypeStruct((_ES, D), jnp.float32),
    )(x, rbf, W_rbf)


_NC = 2
_NS = 16
_NW = _NC * _NS
_EW = _ES // _NW
_CH = 128
_CF = _EW // _CH
_TAIL = _EW - _CF * _CH
N_PAD = 10112
_RPW = N_PAD // _NS

assert _CF >= 2 and _TAIL % 8 == 0 and _EW % 8 == 0

_sc_mesh = plsc.VectorSubcoreMesh(core_axis_name="c", subcore_axis_name="s")


def _make_scatter(sl):
    idx_base0 = sl * _ES

    @functools.partial(
        pl.kernel,
        out_type=jax.ShapeDtypeStruct((_NC, N_PAD, D), jnp.float32),
        mesh=_sc_mesh,
        scratch_types=[
            pltpu.VMEM_SHARED((N_PAD, D), jnp.float32),
            pltpu.VMEM((_CH, D), jnp.float32),
            pltpu.VMEM((_CH, D), jnp.float32),
            pltpu.VMEM((_CH,), jnp.int32),
            pltpu.VMEM((_CH,), jnp.int32),
            pltpu.VMEM((_TAIL, D), jnp.float32),
            pltpu.VMEM((_TAIL,), jnp.int32),
            pltpu.SemaphoreType.DMA,
            pltpu.SemaphoreType.DMA,
            pltpu.SemaphoreType.DMA,
            pltpu.SemaphoreType.DMA,
        ],
    )
    def _scatter_stage(h_hbm, idx_hbm, zeros_hbm, out_hbm,
                       acc, b0, b1, i0, i1, bt, it, sr0, sr1, si0, si1):
        c = lax.axis_index("c")
        s = lax.axis_index("s")
        hbase = (c * _NS + s) * _EW
        ibase = idx_base0 + hbase

        pltpu.sync_copy(zeros_hbm.at[pl.ds(s * _RPW, _RPW)],
                        acc.at[pl.ds(s * _RPW, _RPW)])
        plsc.subcore_barrier()

        def start(chunk, rb, ib, sr, si):
            pltpu.async_copy(h_hbm.at[pl.ds(hbase + chunk * _CH, _CH)], rb, sr)
            pltpu.async_copy(idx_hbm.at[pl.ds(ibase + chunk * _CH, _CH)],
                             ib, si)

        def wait(rb, ib, sr, si):
            pltpu.make_async_copy(h_hbm.at[pl.ds(0, _CH)], rb, sr).wait()
            pltpu.make_async_copy(idx_hbm.at[pl.ds(0, _CH)], ib, si).wait()

        start(0, b0, i0, sr0, si0)
        start(1, b1, i1, sr1, si1)
        nloop = (_CF - 2) // 2

        def body(j, carry):
            wait(b0, i0, sr0, si0)
            pltpu.sync_copy(b0, acc.at[i0], add=True)
            start(2 * j + 2, b0, i0, sr0, si0)
            wait(b1, i1, sr1, si1)
            pltpu.sync_copy(b1, acc.at[i1], add=True)
            start(2 * j + 3, b1, i1, sr1, si1)
            return carry

        lax.fori_loop(0, nloop, body, 0)

        wait(b0, i0, sr0, si0)
        pltpu.sync_copy(b0, acc.at[i0], add=True)
        wait(b1, i1, sr1, si1)
        pltpu.sync_copy(b1, acc.at[i1], add=True)

        if _CF % 2:
            start(_CF - 1, b0, i0, sr0, si0)
            wait(b0, i0, sr0, si0)
            pltpu.sync_copy(b0, acc.at[i0], add=True)

        if _TAIL:
            toff = _CF * _CH
            pltpu.sync_copy(h_hbm.at[pl.ds(hbase + toff, _TAIL)], bt)
            pltpu.sync_copy(idx_hbm.at[pl.ds(ibase + toff, _TAIL)], it)
            pltpu.sync_copy(bt, acc.at[it], add=True)

        plsc.subcore_barrier()
        pltpu.sync_copy(acc.at[pl.ds(s * _RPW, _RPW)],
                        out_hbm.at[c, pl.ds(s * _RPW, _RPW)])

    return _scatter_stage


_scatter_stages = [_make_scatter(sl) for sl in range(_NSLICE)]

_NODE_BLK = 1264


def _mlp_body(*refs):
    p_refs = refs[:_NSLICE]
    w1_ref, b1_ref, w2_ref, b2_ref, w3_ref, b3_ref, wo_ref, out_ref = \
        refs[_NSLICE:]
    h = p_refs[0][0] + p_refs[0][1]
    for p in p_refs[1:]:
        h = h + p[0] + p[1]
    h = jax.nn.silu(jnp.dot(h, w1_ref[...],
                            preferred_element_type=jnp.float32) + b1_ref[...])
    h = jax.nn.silu(jnp.dot(h, w2_ref[...],
                            preferred_element_type=jnp.float32) + b2_ref[...])
    h = jax.nn.silu(jnp.dot(h, w3_ref[...],
                            preferred_element_type=jnp.float32) + b3_ref[...])
    out_ref[...] = jnp.dot(h, wo_ref[...], preferred_element_type=jnp.float32)


def _mlp_stage(partials, W1, b1, W2, b2, W3, b3, W_out_pad):
    grid = (N_PAD // _NODE_BLK,)
    full = lambda i: (0, 0)
    return pl.pallas_call(
        _mlp_body,
        grid=grid,
        in_specs=[pl.BlockSpec((_NC, _NODE_BLK, D), lambda i: (0, i, 0))
                  for _ in range(_NSLICE)] + [
            pl.BlockSpec((D, D), full),
            pl.BlockSpec((1, D), full),
            pl.BlockSpec((D, D), full),
            pl.BlockSpec((1, D), full),
            pl.BlockSpec((D, D), full),
            pl.BlockSpec((1, D), full),
            pl.BlockSpec((D, D), full),
        ],
        out_specs=pl.BlockSpec((_NODE_BLK, D), lambda i: (i, 0)),
        out_shape=jax.ShapeDtypeStruct((N_PAD, D), jnp.float32),
    )(*partials, W1, b1, W2, b2, W3, b3, W_out_pad)


def kernel(x, rbf, idnb_i, n_atoms, W_rbf, W1, b1, W2, b2, W3, b3, W_out):
    del n_atoms
    zeros = jnp.zeros((N_PAD, D), dtype=jnp.float32)
    partials = []
    for sl in range(_NSLICE):
        h_s = _edge_stage(x, rbf, W_rbf, sl)
        partials.append(_scatter_stages[sl](h_s, idnb_i, zeros))
    W_out_pad = jnp.pad(W_out, ((0, 0), (0, D - T)))
    out = _mlp_stage(partials, W1, b1.reshape(1, D), W2, b2.reshape(1, D),
                     W3, b3.reshape(1, D), W_out_pad)
    return out[:N, :T]


# speedup vs baseline: 3.1958x; 1.0238x over previous
import functools

import jax
import jax.numpy as jnp
from jax import lax
from jax.experimental import pallas as pl
from jax.experimental.pallas import tpu as pltpu
from jax.experimental.pallas import tpu_sc as plsc

E = 320000
N = 10000
D = 128
R = 16
T = 12

_SLICES = (161280, 158720)
_NSLICE = len(_SLICES)
_SLICE_OFF = tuple(sum(_SLICES[:i]) for i in range(_NSLICE))
assert sum(_SLICES) == E

_EB = 1280


def _edge_body(xl_ref, xh_ref, rl_ref, rh_ref, w_ref, out_ref):
    def half(rbf_ref, x_ref):
        g = jnp.dot(rbf_ref[...], w_ref[...],
                    preferred_element_type=jnp.float32)
        hbf = (g * x_ref[...]).astype(jnp.bfloat16)
        a = lax.bitcast_convert_type(hbf[:, :64], jnp.uint16)
        b = lax.bitcast_convert_type(hbf[:, 64:], jnp.uint16)
        word = a.astype(jnp.uint32) | (b.astype(jnp.uint32) << 16)
        return lax.bitcast_convert_type(word, jnp.int32)

    out_ref[...] = jnp.concatenate(
        [half(rl_ref, xl_ref), half(rh_ref, xh_ref)], axis=1)


def _edge_stage(x, rbf, W_rbf, sl):
    es = _SLICES[sl]
    bps = es // (2 * _EB)
    lo = _SLICE_OFF[sl] // _EB
    hi = lo + bps
    return pl.pallas_call(
        _edge_body,
        grid=(bps,),
        in_specs=[
            pl.BlockSpec((_EB, D), lambda i: (i + lo, 0)),
            pl.BlockSpec((_EB, D), lambda i: (i + hi, 0)),
            pl.BlockSpec((_EB, R), lambda i: (i + lo, 0)),
            pl.BlockSpec((_EB, R), lambda i: (i + hi, 0)),
            pl.BlockSpec((R, D), lambda i: (0, 0)),
        ],
        out_specs=pl.BlockSpec((_EB, D), lambda i: (i, 0)),
        out_shape=jax.ShapeDtypeStruct((es // 2, D), jnp.int32),
    )(x, x, rbf, rbf, W_rbf)


_NC = 2
_NS = 16
_NW = _NC * _NS
_CH = 64
N_PAD = 10112
_RPW = N_PAD // _NS

_sc_mesh = plsc.VectorSubcoreMesh(core_axis_name="c", subcore_axis_name="s")


def _make_scatter(sl, chained):
    es = _SLICES[sl]
    half = es // 2
    rw = half // _NW
    cf = rw // _CH
    tail = rw - cf * _CH
    idx_base0 = _SLICE_OFF[sl]
    ic = _NC if chained else 1
    assert cf >= 4 and rw % 8 == 0 and tail % 8 == 0

    scratch = [
        pltpu.VMEM_SHARED((N_PAD, D), jnp.float32),
        pltpu.SemaphoreType.DMA,
        pltpu.SemaphoreType.DMA,
        pltpu.SemaphoreType.DMA,
        pltpu.SemaphoreType.DMA,
        pltpu.SemaphoreType.DMA,
        pltpu.SemaphoreType.DMA,
        pltpu.SemaphoreType.DMA,
        pltpu.SemaphoreType.DMA,
    ]

    @functools.partial(
        pl.kernel,
        out_type=jax.ShapeDtypeStruct((_NC, N_PAD, D), jnp.float32),
        mesh=_sc_mesh,
        scratch_types=scratch,
    )
    def _scatter_stage(h_hbm, idx_hbm, init_hbm, out_hbm, acc,
                       sw0, sw1, sil0, sih0, sil1, sih1, ss0, ss1):
      def _body(hb0, hb1, f0, f1, i0, i1, j0, j1, it_):
        c = lax.axis_index("c")
        s = lax.axis_index("s")
        rbase = (c * _NS + s) * rw
        ibase_lo = idx_base0 + rbase
        ibase_hi = idx_base0 + half + rbase
        HB, F, I, J = (hb0, hb1), (f0, f1), (i0, i1), (j0, j1)
        SW, SIL, SIH, SS = (sw0, sw1), (sil0, sil1), (sih0, sih1), (ss0, ss1)

        pltpu.sync_copy(init_hbm.at[c * (ic - 1), pl.ds(s * _RPW, _RPW)],
                        acc.at[pl.ds(s * _RPW, _RPW)])
        plsc.subcore_barrier()

        def load(k, b):
            pltpu.async_copy(h_hbm.at[pl.ds(rbase + k * _CH, _CH)],
                             HB[b], SW[b])
            pltpu.async_copy(idx_hbm.at[pl.ds(ibase_lo + k * _CH, _CH)],
                             I[b].at[pl.ds(0, _CH)], SIL[b])
            pltpu.async_copy(idx_hbm.at[pl.ds(ibase_hi + k * _CH, _CH)],
                             I[b].at[pl.ds(_CH, _CH)], SIH[b])

        def wload(b):
            pltpu.make_async_copy(h_hbm.at[pl.ds(0, _CH)], HB[b],
                                  SW[b]).wait()
            pltpu.make_async_copy(idx_hbm.at[pl.ds(0, _CH)],
                                  I[b].at[pl.ds(0, _CH)], SIL[b]).wait()
            pltpu.make_async_copy(idx_hbm.at[pl.ds(0, _CH)],
                                  I[b].at[pl.ds(0, _CH)], SIH[b]).wait()

        def conv(hb, f, nrows):
            @plsc.parallel_loop(0, nrows, step=1, unroll=4)
            def _row(r):
                for widx, roff in ((0, 0), (64, nrows)):
                    for g in range(4):
                        v = hb[r, pl.ds(widx + g * 16, 16)]
                        f[roff + r, pl.ds(g * 16, 16)] = \
                            lax.bitcast_convert_type(v << 16, jnp.float32)
                        f[roff + r, pl.ds(64 + g * 16, 16)] = \
                            lax.bitcast_convert_type(
                                v & jnp.int32(-65536), jnp.float32)

        def wait_scat(b):
            pltpu.make_async_copy(F[b], acc.at[J[b]], SS[b]).wait()

        def proc(b, wait_prev=True):
            wload(b)
            for g in range(2 * _CH // 16):
                J[b][pl.ds(g * 16, 16)] = I[b][pl.ds(g * 16, 16)]
            conv(HB[b], F[b], _CH)
            if wait_prev:
                wait_scat(1 - b)
            pltpu.async_copy(F[b], acc.at[J[b]], SS[b], add=True)

        load(0, 0)
        load(1, 1)
        proc(0, wait_prev=False)
        load(2, 0)

        np_steady = (cf - 3) // 2

        def pair(t, carry):
            proc(1)
            load(2 * t + 3, 1)
            proc(0)
            load(2 * t + 4, 0)
            return carry

        lax.fori_loop(0, np_steady, pair, 0)

        loaded = 2 * np_steady + 2
        for k in range(2 * np_steady + 1, cf):
            proc(k % 2)
            nxt = k + 2
            if nxt < cf and nxt > loaded:
                load(nxt, nxt % 2)
                loaded = nxt
        wait_scat((cf - 1) % 2)

        if tail:
            toff = cf * _CH
            pltpu.sync_copy(h_hbm.at[pl.ds(rbase + toff, tail)],
                            hb0.at[pl.ds(0, tail)])
            pltpu.sync_copy(idx_hbm.at[pl.ds(ibase_lo + toff, tail)],
                            it_.at[pl.ds(0, tail)])
            pltpu.sync_copy(idx_hbm.at[pl.ds(ibase_hi + toff, tail)],
                            it_.at[pl.ds(tail, tail)])
            conv(hb0, f0, tail)
            pltpu.sync_copy(f0.at[pl.ds(0, 2 * tail)], acc.at[it_], add=True)

        plsc.subcore_barrier()
        pltpu.sync_copy(acc.at[pl.ds(s * _RPW, _RPW)],
                        out_hbm.at[c, pl.ds(s * _RPW, _RPW)])

      pl.run_scoped(
          _body,
          pltpu.VMEM((_CH, D), jnp.int32),
          pltpu.VMEM((_CH, D), jnp.int32),
          pltpu.VMEM((2 * _CH, D), jnp.float32),
          pltpu.VMEM((2 * _CH, D), jnp.float32),
          pltpu.VMEM((2 * _CH,), jnp.int32),
          pltpu.VMEM((2 * _CH,), jnp.int32),
          pltpu.VMEM((2 * _CH,), jnp.int32),
          pltpu.VMEM((2 * _CH,), jnp.int32),
          pltpu.VMEM((2 * max(tail, 8),), jnp.int32),
      )

    return _scatter_stage


_scatter_stages = [_make_scatter(sl, sl > 0) for sl in range(_NSLICE)]

_NODE_BLK = 1264


def _mlp_body(p_ref, w1_ref, b1_ref, w2_ref, b2_ref, w3_ref, b3_ref, wo_ref,
              out_ref):
    h = p_ref[0] + p_ref[1]
    h = jax.nn.silu(jnp.dot(h, w1_ref[...],
                            preferred_element_type=jnp.float32) + b1_ref[...])
    h = jax.nn.silu(jnp.dot(h, w2_ref[...],
                            preferred_element_type=jnp.float32) + b2_ref[...])
    h = jax.nn.silu(jnp.dot(h, w3_ref[...],
                            preferred_element_type=jnp.float32) + b3_ref[...])
    out_ref[...] = jnp.dot(h, wo_ref[...], preferred_element_type=jnp.float32)


def _mlp_stage(partials, W1, b1, W2, b2, W3, b3, W_out_pad):
    grid = (N_PAD // _NODE_BLK,)
    full = lambda i: (0, 0)
    return pl.pallas_call(
        _mlp_body,
        grid=grid,
        in_specs=[
            pl.BlockSpec((_NC, _NODE_BLK, D), lambda i: (0, i, 0)),
            pl.BlockSpec((D, D), full),
            pl.BlockSpec((1, D), full),
            pl.BlockSpec((D, D), full),
            pl.BlockSpec((1, D), full),
            pl.BlockSpec((D, D), full),
            pl.BlockSpec((1, D), full),
            pl.BlockSpec((D, D), full),
        ],
        out_specs=pl.BlockSpec((_NODE_BLK, D), lambda i: (i, 0)),
        out_shape=jax.ShapeDtypeStruct((N_PAD, D), jnp.float32),
    )(partials, W1, b1, W2, b2, W3, b3, W_out_pad)


def kernel(x, rbf, idnb_i, n_atoms, W_rbf, W1, b1, W2, b2, W3, b3, W_out):
    del n_atoms
    zeros = jnp.zeros((1, N_PAD, D), dtype=jnp.float32)
    h0 = _edge_stage(x, rbf, W_rbf, 0)
    p0 = _scatter_stages[0](h0, idnb_i, zeros)
    h1 = _edge_stage(x, rbf, W_rbf, 1)
    p1 = _scatter_stages[1](h1, idnb_i, p0)
    W_out_pad = jnp.pad(W_out, ((0, 0), (0, D - T)))
    out = _mlp_stage(p1, W1, b1.reshape(1, D), W2, b2.reshape(1, D),
                     W3, b3.reshape(1, D), W_out_pad)
    return out[:N, :T]


# speedup vs baseline: 3.4867x; 1.0910x over previous
import functools

import jax
import jax.numpy as jnp
from jax import lax
from jax.experimental import pallas as pl
from jax.experimental.pallas import tpu as pltpu
from jax.experimental.pallas import tpu_sc as plsc

E = 320000
N = 10000
D = 128
R = 16
T = 12

_SLICES = (192000, 128000)
_NSLICE = len(_SLICES)
_SLICE_OFF = tuple(sum(_SLICES[:i]) for i in range(_NSLICE))
assert sum(_SLICES) == E

def _make_edge_body(rw):
    def _edge_body(x_ref, rbf_ref, w_ref, out_ref):
        g = jnp.dot(rbf_ref[...], w_ref[...],
                    preferred_element_type=jnp.float32)
        hbf = (g * x_ref[...]).astype(jnp.bfloat16)

        def pack(m):
            a = lax.bitcast_convert_type(m[:, :64], jnp.uint16)
            b = lax.bitcast_convert_type(m[:, 64:], jnp.uint16)
            word = a.astype(jnp.uint32) | (b.astype(jnp.uint32) << 16)
            return lax.bitcast_convert_type(word, jnp.int32)

        out_ref[...] = jnp.concatenate(
            [pack(hbf[:rw]), pack(hbf[rw:])], axis=1)
    return _edge_body


def _edge_stage(x, rbf, W_rbf, sl):
    es = _SLICES[sl]
    blk = es // 32
    rw = blk // 2
    off = _SLICE_OFF[sl] // blk
    return pl.pallas_call(
        _make_edge_body(rw),
        grid=(32,),
        in_specs=[
            pl.BlockSpec((blk, D), lambda i: (i + off, 0)),
            pl.BlockSpec((blk, R), lambda i: (i + off, 0)),
            pl.BlockSpec((R, D), lambda i: (0, 0)),
        ],
        out_specs=pl.BlockSpec((rw, D), lambda i: (i, 0)),
        out_shape=jax.ShapeDtypeStruct((es // 2, D), jnp.int32),
    )(x, rbf, W_rbf)


_NC = 2
_NS = 16
_NW = _NC * _NS
_CH = 64
N_PAD = 10112
_RPW = N_PAD // _NS

_sc_mesh = plsc.VectorSubcoreMesh(core_axis_name="c", subcore_axis_name="s")


def _make_scatter(sl, chained):
    es = _SLICES[sl]
    rw = es // 2 // _NW
    cf = rw // _CH
    tail = rw - cf * _CH
    idx_base0 = _SLICE_OFF[sl]
    ic = _NC if chained else 1
    assert cf >= 4 and rw % 8 == 0 and tail % 8 == 0

    scratch = [
        pltpu.VMEM_SHARED((N_PAD, D), jnp.float32),
        pltpu.SemaphoreType.DMA,
        pltpu.SemaphoreType.DMA,
        pltpu.SemaphoreType.DMA,
        pltpu.SemaphoreType.DMA,
        pltpu.SemaphoreType.DMA,
        pltpu.SemaphoreType.DMA,
        pltpu.SemaphoreType.DMA,
        pltpu.SemaphoreType.DMA,
    ]

    @functools.partial(
        pl.kernel,
        out_type=jax.ShapeDtypeStruct((_NC, N_PAD, D), jnp.float32),
        mesh=_sc_mesh,
        scratch_types=scratch,
    )
    def _scatter_stage(h_hbm, idx_hbm, init_hbm, out_hbm, acc,
                       sw0, sw1, sil0, sih0, sil1, sih1, ss0, ss1):
      def _body(hb0, hb1, f0, f1, i0, i1, j0, j1, it_):
        c = lax.axis_index("c")
        s = lax.axis_index("s")
        rbase = (c * _NS + s) * rw
        ibase_lo = idx_base0 + 2 * rbase
        ibase_hi = ibase_lo + rw
        HB, F, I, J = (hb0, hb1), (f0, f1), (i0, i1), (j0, j1)
        SW, SIL, SIH, SS = (sw0, sw1), (sil0, sil1), (sih0, sih1), (ss0, ss1)

        pltpu.sync_copy(init_hbm.at[c * (ic - 1), pl.ds(s * _RPW, _RPW)],
                        acc.at[pl.ds(s * _RPW, _RPW)])
        plsc.subcore_barrier()

        def load(k, b):
            pltpu.async_copy(h_hbm.at[pl.ds(rbase + k * _CH, _CH)],
                             HB[b], SW[b])
            pltpu.async_copy(idx_hbm.at[pl.ds(ibase_lo + k * _CH, _CH)],
                             I[b].at[pl.ds(0, _CH)], SIL[b])
            pltpu.async_copy(idx_hbm.at[pl.ds(ibase_hi + k * _CH, _CH)],
                             I[b].at[pl.ds(_CH, _CH)], SIH[b])

        def wload(b):
            pltpu.make_async_copy(h_hbm.at[pl.ds(0, _CH)], HB[b],
                                  SW[b]).wait()
            pltpu.make_async_copy(idx_hbm.at[pl.ds(0, _CH)],
                                  I[b].at[pl.ds(0, _CH)], SIL[b]).wait()
            pltpu.make_async_copy(idx_hbm.at[pl.ds(0, _CH)],
                                  I[b].at[pl.ds(0, _CH)], SIH[b]).wait()

        def conv(hb, f, nrows):
            @plsc.parallel_loop(0, nrows, step=1, unroll=4)
            def _row(r):
                for widx, roff in ((0, 0), (64, nrows)):
                    for g in range(4):
                        v = hb[r, pl.ds(widx + g * 16, 16)]
                        f[roff + r, pl.ds(g * 16, 16)] = \
                            lax.bitcast_convert_type(v << 16, jnp.float32)
                        f[roff + r, pl.ds(64 + g * 16, 16)] = \
                            lax.bitcast_convert_type(
                                v & jnp.int32(-65536), jnp.float32)

        def wait_scat(b):
            pltpu.make_async_copy(F[b], acc.at[J[b]], SS[b]).wait()

        def proc(b, wait_prev=True):
            wload(b)
            for g in range(2 * _CH // 16):
                J[b][pl.ds(g * 16, 16)] = I[b][pl.ds(g * 16, 16)]
            conv(HB[b], F[b], _CH)
            if wait_prev:
                wait_scat(1 - b)
            pltpu.async_copy(F[b], acc.at[J[b]], SS[b], add=True)

        load(0, 0)
        load(1, 1)
        proc(0, wait_prev=False)
        load(2, 0)

        np_steady = (cf - 3) // 2

        def pair(t, carry):
            proc(1)
            load(2 * t + 3, 1)
            proc(0)
            load(2 * t + 4, 0)
            return carry

        lax.fori_loop(0, np_steady, pair, 0)

        loaded = 2 * np_steady + 2
        for k in range(2 * np_steady + 1, cf):
            proc(k % 2)
            nxt = k + 2
            if nxt < cf and nxt > loaded:
                load(nxt, nxt % 2)
                loaded = nxt
        wait_scat((cf - 1) % 2)

        if tail:
            toff = cf * _CH
            pltpu.sync_copy(h_hbm.at[pl.ds(rbase + toff, tail)],
                            hb0.at[pl.ds(0, tail)])
            pltpu.sync_copy(idx_hbm.at[pl.ds(ibase_lo + toff, tail)],
                            it_.at[pl.ds(0, tail)])
            pltpu.sync_copy(idx_hbm.at[pl.ds(ibase_hi + toff, tail)],
                            it_.at[pl.ds(tail, tail)])
            conv(hb0, f0, tail)
            pltpu.sync_copy(f0.at[pl.ds(0, 2 * tail)], acc.at[it_], add=True)

        plsc.subcore_barrier()
        pltpu.sync_copy(acc.at[pl.ds(s * _RPW, _RPW)],
                        out_hbm.at[c, pl.ds(s * _RPW, _RPW)])

      pl.run_scoped(
          _body,
          pltpu.VMEM((_CH, D), jnp.int32),
          pltpu.VMEM((_CH, D), jnp.int32),
          pltpu.VMEM((2 * _CH, D), jnp.float32),
          pltpu.VMEM((2 * _CH, D), jnp.float32),
          pltpu.VMEM((2 * _CH,), jnp.int32),
          pltpu.VMEM((2 * _CH,), jnp.int32),
          pltpu.VMEM((2 * _CH,), jnp.int32),
          pltpu.VMEM((2 * _CH,), jnp.int32),
          pltpu.VMEM((2 * max(tail, 8),), jnp.int32),
      )

    return _scatter_stage


_scatter_stages = [_make_scatter(sl, sl > 0) for sl in range(_NSLICE)]

_NODE_BLK = 1264


def _mlp_body(p_ref, w1_ref, b1_ref, w2_ref, b2_ref, w3_ref, b3_ref, wo_ref,
              out_ref):
    h = p_ref[0] + p_ref[1]
    h = jax.nn.silu(jnp.dot(h, w1_ref[...],
                            preferred_element_type=jnp.float32) + b1_ref[...])
    h = jax.nn.silu(jnp.dot(h, w2_ref[...],
                            preferred_element_type=jnp.float32) + b2_ref[...])
    h = jax.nn.silu(jnp.dot(h, w3_ref[...],
                            preferred_element_type=jnp.float32) + b3_ref[...])
    out_ref[...] = jnp.dot(h, wo_ref[...], preferred_element_type=jnp.float32)


def _mlp_stage(partials, W1, b1, W2, b2, W3, b3, W_out_pad):
    grid = (N_PAD // _NODE_BLK,)
    full = lambda i: (0, 0)
    return pl.pallas_call(
        _mlp_body,
        grid=grid,
        in_specs=[
            pl.BlockSpec((_NC, _NODE_BLK, D), lambda i: (0, i, 0)),
            pl.BlockSpec((D, D), full),
            pl.BlockSpec((1, D), full),
            pl.BlockSpec((D, D), full),
            pl.BlockSpec((1, D), full),
            pl.BlockSpec((D, D), full),
            pl.BlockSpec((1, D), full),
            pl.BlockSpec((D, D), full),
        ],
        out_specs=pl.BlockSpec((_NODE_BLK, D), lambda i: (i, 0)),
        out_shape=jax.ShapeDtypeStruct((N_PAD, D), jnp.float32),
    )(partials, W1, b1, W2, b2, W3, b3, W_out_pad)


def kernel(x, rbf, idnb_i, n_atoms, W_rbf, W1, b1, W2, b2, W3, b3, W_out):
    del n_atoms
    zeros = jnp.zeros((1, N_PAD, D), dtype=jnp.float32)
    h0 = _edge_stage(x, rbf, W_rbf, 0)
    p0 = _scatter_stages[0](h0, idnb_i, zeros)
    h1 = _edge_stage(x, rbf, W_rbf, 1)
    p1 = _scatter_stages[1](h1, idnb_i, p0)
    W_out_pad = jnp.pad(W_out, ((0, 0), (0, D - T)))
    out = _mlp_stage(p1, W1, b1.reshape(1, D), W2, b2.reshape(1, D),
                     W3, b3.reshape(1, D), W_out_pad)
    return out[:N, :T]


# speedup vs baseline: 4.3864x; 1.2580x over previous
import functools

import jax
import jax.numpy as jnp
from jax import lax
from jax.experimental import pallas as pl
from jax.experimental.pallas import tpu as pltpu
from jax.experimental.pallas import tpu_sc as plsc

E = 320000
N = 10000
D = 128
R = 16
T = 12

_SLICES = (192000, 128000)
_NSLICE = len(_SLICES)
_SLICE_OFF = tuple(sum(_SLICES[:i]) for i in range(_NSLICE))
assert sum(_SLICES) == E

_EB = 1280


def _edge_body(xl_ref, xh_ref, rl_ref, rh_ref, w_ref, out_ref):
    def half(rbf_t_ref, x_ref):
        g = lax.dot_general(rbf_t_ref[...], w_ref[...],
                            (((0,), (0,)), ((), ())),
                            preferred_element_type=jnp.float32)
        m = (g * x_ref[...]).astype(jnp.bfloat16)
        a = lax.bitcast_convert_type(m[:, :64], jnp.uint16)
        b = lax.bitcast_convert_type(m[:, 64:], jnp.uint16)
        word = a.astype(jnp.uint32) | (b.astype(jnp.uint32) << 16)
        return lax.bitcast_convert_type(word, jnp.int32)

    out_ref[...] = jnp.concatenate(
        [half(rl_ref, xl_ref), half(rh_ref, xh_ref)], axis=1)


def _edge_stage(x, rbf, W_rbf, sl):
    es = _SLICES[sl]
    bps = es // (2 * _EB)
    lo = _SLICE_OFF[sl] // _EB
    hi = lo + bps
    return pl.pallas_call(
        _edge_body,
        grid=(bps,),
        in_specs=[
            pl.BlockSpec((_EB, D), lambda i: (i + lo, 0)),
            pl.BlockSpec((_EB, D), lambda i: (i + hi, 0)),
            pl.BlockSpec((R, _EB), lambda i: (0, i + lo)),
            pl.BlockSpec((R, _EB), lambda i: (0, i + hi)),
            pl.BlockSpec((R, D), lambda i: (0, 0)),
        ],
        out_specs=pl.BlockSpec((_EB, D), lambda i: (i, 0)),
        out_shape=jax.ShapeDtypeStruct((es // 2, D), jnp.int32),
    )(x, x, rbf.T, rbf.T, W_rbf)


_NC = 2
_NS = 16
_NW = _NC * _NS
_CH = 64
N_PAD = 10112
_RPW = N_PAD // _NS

_sc_mesh = plsc.VectorSubcoreMesh(core_axis_name="c", subcore_axis_name="s")


def _make_scatter(sl, chained):
    es = _SLICES[sl]
    rw = es // 2 // _NW
    cf = rw // _CH
    tail = rw - cf * _CH
    idx_base0 = _SLICE_OFF[sl]
    ic = _NC if chained else 1
    assert cf >= 4 and rw % 8 == 0 and tail % 8 == 0

    scratch = [
        pltpu.VMEM_SHARED((N_PAD, D), jnp.float32),
        pltpu.SemaphoreType.DMA,
        pltpu.SemaphoreType.DMA,
        pltpu.SemaphoreType.DMA,
        pltpu.SemaphoreType.DMA,
        pltpu.SemaphoreType.DMA,
        pltpu.SemaphoreType.DMA,
        pltpu.SemaphoreType.DMA,
        pltpu.SemaphoreType.DMA,
    ]

    @functools.partial(
        pl.kernel,
        out_type=jax.ShapeDtypeStruct((_NC, N_PAD, D), jnp.float32),
        mesh=_sc_mesh,
        scratch_types=scratch,
    )
    def _scatter_stage(h_hbm, idx_hbm, init_hbm, out_hbm, acc,
                       sw0, sw1, sil0, sih0, sil1, sih1, ss0, ss1):
      def _body(hb0, hb1, f0, f1, i0, i1, j0, j1, it_):
        c = lax.axis_index("c")
        s = lax.axis_index("s")
        rbase = (c * _NS + s) * rw
        ibase_lo = idx_base0 + rbase
        ibase_hi = idx_base0 + es // 2 + rbase
        HB, F, I, J = (hb0, hb1), (f0, f1), (i0, i1), (j0, j1)
        SW, SIL, SIH, SS = (sw0, sw1), (sil0, sil1), (sih0, sih1), (ss0, ss1)

        pltpu.sync_copy(init_hbm.at[c * (ic - 1), pl.ds(s * _RPW, _RPW)],
                        acc.at[pl.ds(s * _RPW, _RPW)])
        plsc.subcore_barrier()

        def load(k, b):
            pltpu.async_copy(h_hbm.at[pl.ds(rbase + k * _CH, _CH)],
                             HB[b], SW[b])
            pltpu.async_copy(idx_hbm.at[pl.ds(ibase_lo + k * _CH, _CH)],
                             I[b].at[pl.ds(0, _CH)], SIL[b])
            pltpu.async_copy(idx_hbm.at[pl.ds(ibase_hi + k * _CH, _CH)],
                             I[b].at[pl.ds(_CH, _CH)], SIH[b])

        def wload(b):
            pltpu.make_async_copy(h_hbm.at[pl.ds(0, _CH)], HB[b],
                                  SW[b]).wait()
            pltpu.make_async_copy(idx_hbm.at[pl.ds(0, _CH)],
                                  I[b].at[pl.ds(0, _CH)], SIL[b]).wait()
            pltpu.make_async_copy(idx_hbm.at[pl.ds(0, _CH)],
                                  I[b].at[pl.ds(0, _CH)], SIH[b]).wait()

        def conv(hb, f, nrows):
            @plsc.parallel_loop(0, nrows, step=1, unroll=4)
            def _row(r):
                for widx, roff in ((0, 0), (64, nrows)):
                    for g in range(4):
                        v = hb[r, pl.ds(widx + g * 16, 16)]
                        f[roff + r, pl.ds(g * 16, 16)] = \
                            lax.bitcast_convert_type(v << 16, jnp.float32)
                        f[roff + r, pl.ds(64 + g * 16, 16)] = \
                            lax.bitcast_convert_type(
                                v & jnp.int32(-65536), jnp.float32)

        def wait_scat(b):
            pltpu.make_async_copy(F[b], acc.at[J[b]], SS[b]).wait()

        def proc(b, wait_prev=True):
            wload(b)
            for g in range(2 * _CH // 16):
                J[b][pl.ds(g * 16, 16)] = I[b][pl.ds(g * 16, 16)]
            conv(HB[b], F[b], _CH)
            if wait_prev:
                wait_scat(1 - b)
            pltpu.async_copy(F[b], acc.at[J[b]], SS[b], add=True)

        load(0, 0)
        load(1, 1)
        proc(0, wait_prev=False)
        load(2, 0)

        np_steady = (cf - 3) // 2

        def pair(t, carry):
            proc(1)
            load(2 * t + 3, 1)
            proc(0)
            load(2 * t + 4, 0)
            return carry

        lax.fori_loop(0, np_steady, pair, 0)

        loaded = 2 * np_steady + 2
        for k in range(2 * np_steady + 1, cf):
            proc(k % 2)
            nxt = k + 2
            if nxt < cf and nxt > loaded:
                load(nxt, nxt % 2)
                loaded = nxt
        wait_scat((cf - 1) % 2)

        if tail:
            toff = cf * _CH
            pltpu.sync_copy(h_hbm.at[pl.ds(rbase + toff, tail)],
                            hb0.at[pl.ds(0, tail)])
            pltpu.sync_copy(idx_hbm.at[pl.ds(ibase_lo + toff, tail)],
                            it_.at[pl.ds(0, tail)])
            pltpu.sync_copy(idx_hbm.at[pl.ds(ibase_hi + toff, tail)],
                            it_.at[pl.ds(tail, tail)])
            conv(hb0, f0, tail)
            pltpu.sync_copy(f0.at[pl.ds(0, 2 * tail)], acc.at[it_], add=True)

        plsc.subcore_barrier()
        pltpu.sync_copy(acc.at[pl.ds(s * _RPW, _RPW)],
                        out_hbm.at[c, pl.ds(s * _RPW, _RPW)])

      pl.run_scoped(
          _body,
          pltpu.VMEM((_CH, D), jnp.int32),
          pltpu.VMEM((_CH, D), jnp.int32),
          pltpu.VMEM((2 * _CH, D), jnp.float32),
          pltpu.VMEM((2 * _CH, D), jnp.float32),
          pltpu.VMEM((2 * _CH,), jnp.int32),
          pltpu.VMEM((2 * _CH,), jnp.int32),
          pltpu.VMEM((2 * _CH,), jnp.int32),
          pltpu.VMEM((2 * _CH,), jnp.int32),
          pltpu.VMEM((2 * max(tail, 8),), jnp.int32),
      )

    return _scatter_stage


_scatter_stages = [_make_scatter(sl, sl > 0) for sl in range(_NSLICE)]

_NODE_BLK = 1264


def _mlp_body(p_ref, w1_ref, b1_ref, w2_ref, b2_ref, w3_ref, b3_ref, wo_ref,
              out_ref):
    h = p_ref[0] + p_ref[1]
    h = jax.nn.silu(jnp.dot(h, w1_ref[...],
                            preferred_element_type=jnp.float32) + b1_ref[...])
    h = jax.nn.silu(jnp.dot(h, w2_ref[...],
                            preferred_element_type=jnp.float32) + b2_ref[...])
    h = jax.nn.silu(jnp.dot(h, w3_ref[...],
                            preferred_element_type=jnp.float32) + b3_ref[...])
    out_ref[...] = jnp.dot(h, wo_ref[...], preferred_element_type=jnp.float32)


def _mlp_stage(partials, W1, b1, W2, b2, W3, b3, W_out_pad):
    grid = (N_PAD // _NODE_BLK,)
    full = lambda i: (0, 0)
    return pl.pallas_call(
        _mlp_body,
        grid=grid,
        in_specs=[
            pl.BlockSpec((_NC, _NODE_BLK, D), lambda i: (0, i, 0)),
            pl.BlockSpec((D, D), full),
            pl.BlockSpec((1, D), full),
            pl.BlockSpec((D, D), full),
            pl.BlockSpec((1, D), full),
            pl.BlockSpec((D, D), full),
            pl.BlockSpec((1, D), full),
            pl.BlockSpec((D, D), full),
        ],
        out_specs=pl.BlockSpec((_NODE_BLK, D), lambda i: (i, 0)),
        out_shape=jax.ShapeDtypeStruct((N_PAD, D), jnp.float32),
    )(partials, W1, b1, W2, b2, W3, b3, W_out_pad)


def kernel(x, rbf, idnb_i, n_atoms, W_rbf, W1, b1, W2, b2, W3, b3, W_out):
    del n_atoms
    zeros = jnp.zeros((1, N_PAD, D), dtype=jnp.float32)
    h0 = _edge_stage(x, rbf, W_rbf, 0)
    p0 = _scatter_stages[0](h0, idnb_i, zeros)
    h1 = _edge_stage(x, rbf, W_rbf, 1)
    p1 = _scatter_stages[1](h1, idnb_i, p0)
    W_out_pad = jnp.pad(W_out, ((0, 0), (0, D - T)))
    out = _mlp_stage(p1, W1, b1.reshape(1, D), W2, b2.reshape(1, D),
                     W3, b3.reshape(1, D), W_out_pad)
    return out[:N, :T]


# speedup vs baseline: 4.4917x; 1.0240x over previous
import functools

import jax
import jax.numpy as jnp
from jax import lax
from jax.experimental import pallas as pl
from jax.experimental.pallas import tpu as pltpu
from jax.experimental.pallas import tpu_sc as plsc

E = 320000
N = 10000
D = 128
R = 16
T = 12

_SLICES = (130560, 104960, 84480)
_NSLICE = len(_SLICES)
_SLICE_OFF = tuple(sum(_SLICES[:i]) for i in range(_NSLICE))
assert sum(_SLICES) == E

_EB = 1280


def _edge_body(xl_ref, xh_ref, rl_ref, rh_ref, w_ref, out_ref):
    def half(rbf_t_ref, x_ref):
        g = lax.dot_general(rbf_t_ref[...], w_ref[...],
                            (((0,), (0,)), ((), ())),
                            preferred_element_type=jnp.float32)
        m = (g * x_ref[...]).astype(jnp.bfloat16)
        a = lax.bitcast_convert_type(m[:, :64], jnp.uint16)
        b = lax.bitcast_convert_type(m[:, 64:], jnp.uint16)
        word = a.astype(jnp.uint32) | (b.astype(jnp.uint32) << 16)
        return lax.bitcast_convert_type(word, jnp.int32)

    out_ref[...] = jnp.concatenate(
        [half(rl_ref, xl_ref), half(rh_ref, xh_ref)], axis=1)


def _edge_stage(x, rbf, W_rbf, sl):
    es = _SLICES[sl]
    bps = es // (2 * _EB)
    lo = _SLICE_OFF[sl] // _EB
    hi = lo + bps
    return pl.pallas_call(
        _edge_body,
        grid=(bps,),
        in_specs=[
            pl.BlockSpec((_EB, D), lambda i: (i + lo, 0)),
            pl.BlockSpec((_EB, D), lambda i: (i + hi, 0)),
            pl.BlockSpec((R, _EB), lambda i: (0, i + lo)),
            pl.BlockSpec((R, _EB), lambda i: (0, i + hi)),
            pl.BlockSpec((R, D), lambda i: (0, 0)),
        ],
        out_specs=pl.BlockSpec((_EB, D), lambda i: (i, 0)),
        out_shape=jax.ShapeDtypeStruct((es // 2, D), jnp.int32),
    )(x, x, rbf.T, rbf.T, W_rbf)


_NC = 2
_NS = 16
_NW = _NC * _NS
_CH = 64
N_PAD = 10112
_RPW = N_PAD // _NS

_sc_mesh = plsc.VectorSubcoreMesh(core_axis_name="c", subcore_axis_name="s")


def _make_scatter(sl, chained):
    es = _SLICES[sl]
    rw = es // 2 // _NW
    cf = rw // _CH
    tail = rw - cf * _CH
    idx_base0 = _SLICE_OFF[sl]
    ic = _NC if chained else 1
    assert cf >= 4 and rw % 8 == 0 and tail % 8 == 0

    scratch = [
        pltpu.VMEM_SHARED((N_PAD, D), jnp.float32),
        pltpu.SemaphoreType.DMA,
        pltpu.SemaphoreType.DMA,
        pltpu.SemaphoreType.DMA,
        pltpu.SemaphoreType.DMA,
        pltpu.SemaphoreType.DMA,
        pltpu.SemaphoreType.DMA,
        pltpu.SemaphoreType.DMA,
        pltpu.SemaphoreType.DMA,
    ]

    @functools.partial(
        pl.kernel,
        out_type=jax.ShapeDtypeStruct((_NC, N_PAD, D), jnp.float32),
        mesh=_sc_mesh,
        scratch_types=scratch,
    )
    def _scatter_stage(h_hbm, idx_hbm, init_hbm, out_hbm, acc,
                       sw0, sw1, sil0, sih0, sil1, sih1, ss0, ss1):
      def _body(hb0, hb1, f0, f1, i0, i1, j0, j1, it_):
        c = lax.axis_index("c")
        s = lax.axis_index("s")
        rbase = (c * _NS + s) * rw
        ibase_lo = idx_base0 + rbase
        ibase_hi = idx_base0 + es // 2 + rbase
        HB, F, I, J = (hb0, hb1), (f0, f1), (i0, i1), (j0, j1)
        SW, SIL, SIH, SS = (sw0, sw1), (sil0, sil1), (sih0, sih1), (ss0, ss1)

        pltpu.sync_copy(init_hbm.at[c * (ic - 1), pl.ds(s * _RPW, _RPW)],
                        acc.at[pl.ds(s * _RPW, _RPW)])
        plsc.subcore_barrier()

        def load(k, b):
            pltpu.async_copy(h_hbm.at[pl.ds(rbase + k * _CH, _CH)],
                             HB[b], SW[b])
            pltpu.async_copy(idx_hbm.at[pl.ds(ibase_lo + k * _CH, _CH)],
                             I[b].at[pl.ds(0, _CH)], SIL[b])
            pltpu.async_copy(idx_hbm.at[pl.ds(ibase_hi + k * _CH, _CH)],
                             I[b].at[pl.ds(_CH, _CH)], SIH[b])

        def wload(b):
            pltpu.make_async_copy(h_hbm.at[pl.ds(0, _CH)], HB[b],
                                  SW[b]).wait()
            pltpu.make_async_copy(idx_hbm.at[pl.ds(0, _CH)],
                                  I[b].at[pl.ds(0, _CH)], SIL[b]).wait()
            pltpu.make_async_copy(idx_hbm.at[pl.ds(0, _CH)],
                                  I[b].at[pl.ds(0, _CH)], SIH[b]).wait()

        def conv(hb, f, nrows):
            @plsc.parallel_loop(0, nrows, step=1, unroll=4)
            def _row(r):
                for widx, roff in ((0, 0), (64, nrows)):
                    for g in range(4):
                        v = hb[r, pl.ds(widx + g * 16, 16)]
                        f[roff + r, pl.ds(g * 16, 16)] = \
                            lax.bitcast_convert_type(v << 16, jnp.float32)
                        f[roff + r, pl.ds(64 + g * 16, 16)] = \
                            lax.bitcast_convert_type(
                                v & jnp.int32(-65536), jnp.float32)

        def wait_scat(b):
            pltpu.make_async_copy(F[b], acc.at[J[b]], SS[b]).wait()

        def proc(b, wait_prev=True):
            wload(b)
            for g in range(2 * _CH // 16):
                J[b][pl.ds(g * 16, 16)] = I[b][pl.ds(g * 16, 16)]
            conv(HB[b], F[b], _CH)
            if wait_prev:
                wait_scat(1 - b)
            pltpu.async_copy(F[b], acc.at[J[b]], SS[b], add=True)

        load(0, 0)
        load(1, 1)
        proc(0, wait_prev=False)
        load(2, 0)

        np_steady = (cf - 3) // 2

        def pair(t, carry):
            proc(1)
            load(2 * t + 3, 1)
            proc(0)
            load(2 * t + 4, 0)
            return carry

        lax.fori_loop(0, np_steady, pair, 0)

        loaded = 2 * np_steady + 2
        for k in range(2 * np_steady + 1, cf):
            proc(k % 2)
            nxt = k + 2
            if nxt < cf and nxt > loaded:
                load(nxt, nxt % 2)
                loaded = nxt
        wait_scat((cf - 1) % 2)

        if tail:
            toff = cf * _CH
            pltpu.sync_copy(h_hbm.at[pl.ds(rbase + toff, tail)],
                            hb0.at[pl.ds(0, tail)])
            pltpu.sync_copy(idx_hbm.at[pl.ds(ibase_lo + toff, tail)],
                            it_.at[pl.ds(0, tail)])
            pltpu.sync_copy(idx_hbm.at[pl.ds(ibase_hi + toff, tail)],
                            it_.at[pl.ds(tail, tail)])
            conv(hb0, f0, tail)
            pltpu.sync_copy(f0.at[pl.ds(0, 2 * tail)], acc.at[it_], add=True)

        plsc.subcore_barrier()
        pltpu.sync_copy(acc.at[pl.ds(s * _RPW, _RPW)],
                        out_hbm.at[c, pl.ds(s * _RPW, _RPW)])

      pl.run_scoped(
          _body,
          pltpu.VMEM((_CH, D), jnp.int32),
          pltpu.VMEM((_CH, D), jnp.int32),
          pltpu.VMEM((2 * _CH, D), jnp.float32),
          pltpu.VMEM((2 * _CH, D), jnp.float32),
          pltpu.VMEM((2 * _CH,), jnp.int32),
          pltpu.VMEM((2 * _CH,), jnp.int32),
          pltpu.VMEM((2 * _CH,), jnp.int32),
          pltpu.VMEM((2 * _CH,), jnp.int32),
          pltpu.VMEM((2 * max(tail, 8),), jnp.int32),
      )

    return _scatter_stage


_scatter_stages = [_make_scatter(sl, sl > 0) for sl in range(_NSLICE)]

_NODE_BLK = 1264


def _mlp_body(p_ref, w1_ref, b1_ref, w2_ref, b2_ref, w3_ref, b3_ref, wo_ref,
              out_ref):
    h = p_ref[0] + p_ref[1]
    h = jax.nn.silu(jnp.dot(h, w1_ref[...],
                            preferred_element_type=jnp.float32) + b1_ref[...])
    h = jax.nn.silu(jnp.dot(h, w2_ref[...],
                            preferred_element_type=jnp.float32) + b2_ref[...])
    h = jax.nn.silu(jnp.dot(h, w3_ref[...],
                            preferred_element_type=jnp.float32) + b3_ref[...])
    out_ref[...] = jnp.dot(h, wo_ref[...], preferred_element_type=jnp.float32)


def _mlp_stage(partials, W1, b1, W2, b2, W3, b3, W_out):
    grid = (N_PAD // _NODE_BLK,)
    full = lambda i: (0, 0)
    return pl.pallas_call(
        _mlp_body,
        grid=grid,
        in_specs=[
            pl.BlockSpec((_NC, _NODE_BLK, D), lambda i: (0, i, 0)),
            pl.BlockSpec((D, D), full),
            pl.BlockSpec((1, D), full),
            pl.BlockSpec((D, D), full),
            pl.BlockSpec((1, D), full),
            pl.BlockSpec((D, D), full),
            pl.BlockSpec((1, D), full),
            pl.BlockSpec((D, T), full),
        ],
        out_specs=pl.BlockSpec((_NODE_BLK, T), lambda i: (i, 0)),
        out_shape=jax.ShapeDtypeStruct((N, T), jnp.float32),
    )(partials, W1, b1, W2, b2, W3, b3, W_out)


def kernel(x, rbf, idnb_i, n_atoms, W_rbf, W1, b1, W2, b2, W3, b3, W_out):
    del n_atoms
    p = jnp.zeros((1, N_PAD, D), dtype=jnp.float32)
    for sl in range(_NSLICE):
        h_s = _edge_stage(x, rbf, W_rbf, sl)
        p = _scatter_stages[sl](h_s, idnb_i, p)
    return _mlp_stage(p, W1, b1.reshape(1, D), W2, b2.reshape(1, D),
                      W3, b3.reshape(1, D), W_out)


# speedup vs baseline: 4.6330x; 1.0315x over previous
import functools

import jax
import jax.numpy as jnp
from jax import lax
from jax.experimental import pallas as pl
from jax.experimental.pallas import tpu as pltpu
from jax.experimental.pallas import tpu_sc as plsc

E = 320000
N = 10000
D = 128
R = 16
T = 12

_SLICES = (125440, 104960, 89600)
_NSLICE = len(_SLICES)
_SLICE_OFF = tuple(sum(_SLICES[:i]) for i in range(_NSLICE))
assert sum(_SLICES) == E

_EB = 1280


def _edge_body(xl_ref, xh_ref, rl_ref, rh_ref, w_ref, out_ref):
    def half(rbf_t_ref, x_ref):
        g = lax.dot_general(rbf_t_ref[...], w_ref[...],
                            (((0,), (0,)), ((), ())),
                            preferred_element_type=jnp.float32)
        m = (g * x_ref[...]).astype(jnp.bfloat16)
        a = lax.bitcast_convert_type(m[:, :64], jnp.uint16)
        b = lax.bitcast_convert_type(m[:, 64:], jnp.uint16)
        word = a.astype(jnp.uint32) | (b.astype(jnp.uint32) << 16)
        return lax.bitcast_convert_type(word, jnp.int32)

    out_ref[...] = jnp.concatenate(
        [half(rl_ref, xl_ref), half(rh_ref, xh_ref)], axis=1)


def _edge_stage(x, rbf, W_rbf, sl):
    es = _SLICES[sl]
    bps = es // (2 * _EB)
    lo = _SLICE_OFF[sl] // _EB
    hi = lo + bps
    return pl.pallas_call(
        _edge_body,
        grid=(bps,),
        in_specs=[
            pl.BlockSpec((_EB, D), lambda i: (i + lo, 0)),
            pl.BlockSpec((_EB, D), lambda i: (i + hi, 0)),
            pl.BlockSpec((R, _EB), lambda i: (0, i + lo)),
            pl.BlockSpec((R, _EB), lambda i: (0, i + hi)),
            pl.BlockSpec((R, D), lambda i: (0, 0)),
        ],
        out_specs=pl.BlockSpec((_EB, D), lambda i: (i, 0)),
        out_shape=jax.ShapeDtypeStruct((es // 2, D), jnp.int32),
    )(x, x, rbf.T, rbf.T, W_rbf)


_NC = 2
_NS = 16
_NW = _NC * _NS
_CH = 64
N_PAD = 10112
_RPW = N_PAD // _NS

_sc_mesh = plsc.VectorSubcoreMesh(core_axis_name="c", subcore_axis_name="s")


def _make_scatter(sl, chained):
    es = _SLICES[sl]
    rw = es // 2 // _NW
    cf = rw // _CH
    tail = rw - cf * _CH
    idx_base0 = _SLICE_OFF[sl]
    ic = _NC if chained else 1
    assert cf >= 4 and rw % 8 == 0 and tail % 8 == 0

    scratch = [
        pltpu.VMEM_SHARED((N_PAD, D), jnp.float32),
        pltpu.SemaphoreType.DMA,
        pltpu.SemaphoreType.DMA,
        pltpu.SemaphoreType.DMA,
        pltpu.SemaphoreType.DMA,
        pltpu.SemaphoreType.DMA,
        pltpu.SemaphoreType.DMA,
        pltpu.SemaphoreType.DMA,
        pltpu.SemaphoreType.DMA,
    ]

    def _impl(h_hbm, idx_hbm, init_hbm, out_hbm, acc,
              sw0, sw1, sil0, sih0, sil1, sih1, ss0, ss1):
      def _body(hb0, hb1, f0, f1, i0, i1, j0, j1, it_):
        c = lax.axis_index("c")
        s = lax.axis_index("s")
        rbase = (c * _NS + s) * rw
        ibase_lo = idx_base0 + rbase
        ibase_hi = idx_base0 + es // 2 + rbase
        HB, F, I, J = (hb0, hb1), (f0, f1), (i0, i1), (j0, j1)
        SW, SIL, SIH, SS = (sw0, sw1), (sil0, sil1), (sih0, sih1), (ss0, ss1)

        if chained:
            pltpu.sync_copy(init_hbm.at[c, pl.ds(s * _RPW, _RPW)],
                            acc.at[pl.ds(s * _RPW, _RPW)])
        else:
            zf = jnp.zeros((16,), jnp.float32)

            @plsc.parallel_loop(0, 2 * _CH, step=1, unroll=8)
            def _zrow(r):
                for gg in range(8):
                    f0[r, pl.ds(gg * 16, 16)] = zf

            nfull = _RPW // (2 * _CH)
            rem = _RPW - nfull * 2 * _CH
            for rep in range(nfull):
                pltpu.sync_copy(
                    f0, acc.at[pl.ds(s * _RPW + rep * 2 * _CH, 2 * _CH)])
            if rem:
                pltpu.sync_copy(
                    f0.at[pl.ds(0, rem)],
                    acc.at[pl.ds(s * _RPW + nfull * 2 * _CH, rem)])
        plsc.subcore_barrier()

        def load(k, b):
            pltpu.async_copy(h_hbm.at[pl.ds(rbase + k * _CH, _CH)],
                             HB[b], SW[b])
            pltpu.async_copy(idx_hbm.at[pl.ds(ibase_lo + k * _CH, _CH)],
                             I[b].at[pl.ds(0, _CH)], SIL[b])
            pltpu.async_copy(idx_hbm.at[pl.ds(ibase_hi + k * _CH, _CH)],
                             I[b].at[pl.ds(_CH, _CH)], SIH[b])

        def wload(b):
            pltpu.make_async_copy(h_hbm.at[pl.ds(0, _CH)], HB[b],
                                  SW[b]).wait()
            pltpu.make_async_copy(idx_hbm.at[pl.ds(0, _CH)],
                                  I[b].at[pl.ds(0, _CH)], SIL[b]).wait()
            pltpu.make_async_copy(idx_hbm.at[pl.ds(0, _CH)],
                                  I[b].at[pl.ds(0, _CH)], SIH[b]).wait()

        def conv(hb, f, nrows):
            @plsc.parallel_loop(0, nrows, step=1, unroll=4)
            def _row(r):
                for widx, roff in ((0, 0), (64, nrows)):
                    for g in range(4):
                        v = hb[r, pl.ds(widx + g * 16, 16)]
                        f[roff + r, pl.ds(g * 16, 16)] = \
                            lax.bitcast_convert_type(v << 16, jnp.float32)
                        f[roff + r, pl.ds(64 + g * 16, 16)] = \
                            lax.bitcast_convert_type(
                                v & jnp.int32(-65536), jnp.float32)

        def wait_scat(b):
            pltpu.make_async_copy(F[b], acc.at[J[b]], SS[b]).wait()

        def proc(b, wait_prev=True):
            wload(b)
            for g in range(2 * _CH // 16):
                J[b][pl.ds(g * 16, 16)] = I[b][pl.ds(g * 16, 16)]
            conv(HB[b], F[b], _CH)
            if wait_prev:
                wait_scat(1 - b)
            pltpu.async_copy(F[b], acc.at[J[b]], SS[b], add=True)

        load(0, 0)
        load(1, 1)
        proc(0, wait_prev=False)
        load(2, 0)

        np_steady = (cf - 3) // 2

        def pair(t, carry):
            proc(1)
            load(2 * t + 3, 1)
            proc(0)
            load(2 * t + 4, 0)
            return carry

        lax.fori_loop(0, np_steady, pair, 0)

        loaded = 2 * np_steady + 2
        for k in range(2 * np_steady + 1, cf):
            proc(k % 2)
            nxt = k + 2
            if nxt < cf and nxt > loaded:
                load(nxt, nxt % 2)
                loaded = nxt
        wait_scat((cf - 1) % 2)

        if tail:
            toff = cf * _CH
            pltpu.sync_copy(h_hbm.at[pl.ds(rbase + toff, tail)],
                            hb0.at[pl.ds(0, tail)])
            pltpu.sync_copy(idx_hbm.at[pl.ds(ibase_lo + toff, tail)],
                            it_.at[pl.ds(0, tail)])
            pltpu.sync_copy(idx_hbm.at[pl.ds(ibase_hi + toff, tail)],
                            it_.at[pl.ds(tail, tail)])
            conv(hb0, f0, tail)
            pltpu.sync_copy(f0.at[pl.ds(0, 2 * tail)], acc.at[it_], add=True)

        plsc.subcore_barrier()
        pltpu.sync_copy(acc.at[pl.ds(s * _RPW, _RPW)],
                        out_hbm.at[c, pl.ds(s * _RPW, _RPW)])

      pl.run_scoped(
          _body,
          pltpu.VMEM((_CH, D), jnp.int32),
          pltpu.VMEM((_CH, D), jnp.int32),
          pltpu.VMEM((2 * _CH, D), jnp.float32),
          pltpu.VMEM((2 * _CH, D), jnp.float32),
          pltpu.VMEM((2 * _CH,), jnp.int32),
          pltpu.VMEM((2 * _CH,), jnp.int32),
          pltpu.VMEM((2 * _CH,), jnp.int32),
          pltpu.VMEM((2 * _CH,), jnp.int32),
          pltpu.VMEM((2 * max(tail, 8),), jnp.int32),
      )

    kw = dict(out_type=jax.ShapeDtypeStruct((_NC, N_PAD, D), jnp.float32),
              mesh=_sc_mesh, scratch_types=scratch)
    if chained:
        @functools.partial(pl.kernel, **kw)
        def _scatter_stage(h_hbm, idx_hbm, init_hbm, out_hbm, acc,
                           sw0, sw1, sil0, sih0, sil1, sih1, ss0, ss1):
            _impl(h_hbm, idx_hbm, init_hbm, out_hbm, acc,
                  sw0, sw1, sil0, sih0, sil1, sih1, ss0, ss1)
    else:
        @functools.partial(pl.kernel, **kw)
        def _scatter_stage(h_hbm, idx_hbm, out_hbm, acc,
                           sw0, sw1, sil0, sih0, sil1, sih1, ss0, ss1):
            _impl(h_hbm, idx_hbm, None, out_hbm, acc,
                  sw0, sw1, sil0, sih0, sil1, sih1, ss0, ss1)

    return _scatter_stage


_scatter_stages = [_make_scatter(sl, sl > 0) for sl in range(_NSLICE)]

_NODE_BLK = 1264


def _mlp_body(p_ref, w1_ref, b1_ref, w2_ref, b2_ref, w3_ref, b3_ref, wo_ref,
              out_ref):
    h = p_ref[0] + p_ref[1]
    h = jax.nn.silu(jnp.dot(h, w1_ref[...],
                            preferred_element_type=jnp.float32) + b1_ref[...])
    h = jax.nn.silu(jnp.dot(h, w2_ref[...],
                            preferred_element_type=jnp.float32) + b2_ref[...])
    h = jax.nn.silu(jnp.dot(h, w3_ref[...],
                            preferred_element_type=jnp.float32) + b3_ref[...])
    out_ref[...] = jnp.dot(h, wo_ref[...], preferred_element_type=jnp.float32)


def _mlp_stage(partials, W1, b1, W2, b2, W3, b3, W_out):
    grid = (N_PAD // _NODE_BLK,)
    full = lambda i: (0, 0)
    return pl.pallas_call(
        _mlp_body,
        grid=grid,
        in_specs=[
            pl.BlockSpec((_NC, _NODE_BLK, D), lambda i: (0, i, 0)),
            pl.BlockSpec((D, D), full),
            pl.BlockSpec((1, D), full),
            pl.BlockSpec((D, D), full),
            pl.BlockSpec((1, D), full),
            pl.BlockSpec((D, D), full),
            pl.BlockSpec((1, D), full),
            pl.BlockSpec((D, T), full),
        ],
        out_specs=pl.BlockSpec((_NODE_BLK, T), lambda i: (i, 0)),
        out_shape=jax.ShapeDtypeStruct((N, T), jnp.float32),
    )(partials, W1, b1, W2, b2, W3, b3, W_out)


def kernel(x, rbf, idnb_i, n_atoms, W_rbf, W1, b1, W2, b2, W3, b3, W_out):
    del n_atoms
    p = None
    for sl in range(_NSLICE):
        h_s = _edge_stage(x, rbf, W_rbf, sl)
        args = (h_s, idnb_i) if p is None else (h_s, idnb_i, p)
        p = _scatter_stages[sl](*args)
    return _mlp_stage(p, W1, b1.reshape(1, D), W2, b2.reshape(1, D),
                      W3, b3.reshape(1, D), W_out)


# speedup vs baseline: 4.6734x; 1.0087x over previous
import functools

import jax
import jax.numpy as jnp
from jax import lax
from jax.experimental import pallas as pl
from jax.experimental.pallas import tpu as pltpu
from jax.experimental.pallas import tpu_sc as plsc

E = 320000
N = 10000
D = 128
R = 16
T = 12

_SLICES = (125440, 104960, 89600)
_NSLICE = len(_SLICES)
_SLICE_OFF = tuple(sum(_SLICES[:i]) for i in range(_NSLICE))
assert sum(_SLICES) == E

_EB = 1280


def _edge_body(xl_ref, xh_ref, rl_ref, rh_ref, w_ref, out_ref):
    def half(rbf_t_ref, x_ref):
        g = lax.dot_general(rbf_t_ref[...], w_ref[...],
                            (((0,), (0,)), ((), ())),
                            preferred_element_type=jnp.float32)
        m = (g * x_ref[...]).astype(jnp.bfloat16)
        a = lax.bitcast_convert_type(m[:, :64], jnp.uint16)
        b = lax.bitcast_convert_type(m[:, 64:], jnp.uint16)
        word = a.astype(jnp.uint32) | (b.astype(jnp.uint32) << 16)
        return lax.bitcast_convert_type(word, jnp.int32)

    out_ref[...] = jnp.concatenate(
        [half(rl_ref, xl_ref), half(rh_ref, xh_ref)], axis=1)


def _edge_stage(x, rbf, W_rbf, sl):
    es = _SLICES[sl]
    bps = es // (2 * _EB)
    lo = _SLICE_OFF[sl] // _EB
    hi = lo + bps
    return pl.pallas_call(
        _edge_body,
        grid=(bps,),
        in_specs=[
            pl.BlockSpec((_EB, D), lambda i: (i + lo, 0)),
            pl.BlockSpec((_EB, D), lambda i: (i + hi, 0)),
            pl.BlockSpec((R, _EB), lambda i: (0, i + lo)),
            pl.BlockSpec((R, _EB), lambda i: (0, i + hi)),
            pl.BlockSpec((R, D), lambda i: (0, 0)),
        ],
        out_specs=pl.BlockSpec((_EB, D), lambda i: (i, 0)),
        out_shape=jax.ShapeDtypeStruct((es // 2, D), jnp.int32),
    )(x, x, rbf.T, rbf.T, W_rbf)


_NC = 2
_NS = 16
_NW = _NC * _NS
_CH = 64
N_PAD = 10112
_RPW = N_PAD // _NS

_sc_mesh = plsc.VectorSubcoreMesh(core_axis_name="c", subcore_axis_name="s")


def _make_scatter(sl, chained):
    es = _SLICES[sl]
    rw = es // 2 // _NW
    cf = rw // _CH
    tail = rw - cf * _CH
    idx_base0 = _SLICE_OFF[sl]
    ic = _NC if chained else 1
    assert cf >= 4 and rw % 8 == 0 and tail % 8 == 0

    scratch = [
        pltpu.VMEM_SHARED((N_PAD, D), jnp.float32),
        pltpu.SemaphoreType.DMA,
        pltpu.SemaphoreType.DMA,
        pltpu.SemaphoreType.DMA,
        pltpu.SemaphoreType.DMA,
        pltpu.SemaphoreType.DMA,
        pltpu.SemaphoreType.DMA,
        pltpu.SemaphoreType.DMA,
        pltpu.SemaphoreType.DMA,
    ]

    def _impl(h_hbm, idx_hbm, init_hbm, out_hbm, acc,
              sw0, sw1, sil0, sih0, sil1, sih1, ss0, ss1):
      def _body(hb0, hb1, f0, f1, i0, i1, j0, j1, it_):
        c = lax.axis_index("c")
        s = lax.axis_index("s")
        rbase = (c * _NS + s) * rw
        ibase_lo = idx_base0 + rbase
        ibase_hi = idx_base0 + es // 2 + rbase
        HB, F, I, J = (hb0, hb1), (f0, f1), (i0, i1), (j0, j1)
        SW, SIL, SIH, SS = (sw0, sw1), (sil0, sil1), (sih0, sih1), (ss0, ss1)

        if chained:
            pltpu.sync_copy(init_hbm.at[c, pl.ds(s * _RPW, _RPW)],
                            acc.at[pl.ds(s * _RPW, _RPW)])
        else:
            zf = jnp.zeros((16,), jnp.float32)

            @plsc.parallel_loop(0, 2 * _CH, step=1, unroll=8)
            def _zrow(r):
                for gg in range(8):
                    f0[r, pl.ds(gg * 16, 16)] = zf

            nfull = _RPW // (2 * _CH)
            rem = _RPW - nfull * 2 * _CH
            for rep in range(nfull):
                pltpu.sync_copy(
                    f0, acc.at[pl.ds(s * _RPW + rep * 2 * _CH, 2 * _CH)])
            if rem:
                pltpu.sync_copy(
                    f0.at[pl.ds(0, rem)],
                    acc.at[pl.ds(s * _RPW + nfull * 2 * _CH, rem)])
        plsc.subcore_barrier()

        def load(k, b):
            pltpu.async_copy(h_hbm.at[pl.ds(rbase + k * _CH, _CH)],
                             HB[b], SW[b])
            pltpu.async_copy(idx_hbm.at[pl.ds(ibase_lo + k * _CH, _CH)],
                             I[b].at[pl.ds(0, _CH)], SIL[b])
            pltpu.async_copy(idx_hbm.at[pl.ds(ibase_hi + k * _CH, _CH)],
                             I[b].at[pl.ds(_CH, _CH)], SIH[b])

        def wload(b):
            pltpu.make_async_copy(h_hbm.at[pl.ds(0, _CH)], HB[b],
                                  SW[b]).wait()
            pltpu.make_async_copy(idx_hbm.at[pl.ds(0, _CH)],
                                  I[b].at[pl.ds(0, _CH)], SIL[b]).wait()
            pltpu.make_async_copy(idx_hbm.at[pl.ds(0, _CH)],
                                  I[b].at[pl.ds(0, _CH)], SIH[b]).wait()

        def conv(hb, f, nrows):
            @plsc.parallel_loop(0, nrows, step=1, unroll=4)
            def _row(r):
                for widx, roff in ((0, 0), (64, nrows)):
                    for g in range(4):
                        v = hb[r, pl.ds(widx + g * 16, 16)]
                        f[roff + r, pl.ds(g * 16, 16)] = \
                            lax.bitcast_convert_type(v << 16, jnp.float32)
                        f[roff + r, pl.ds(64 + g * 16, 16)] = \
                            lax.bitcast_convert_type(
                                v & jnp.int32(-65536), jnp.float32)

        def wait_scat(b):
            pltpu.make_async_copy(F[b], acc.at[J[b]], SS[b]).wait()

        def proc(b, wait_prev=True):
            wload(b)
            for g in range(2 * _CH // 16):
                J[b][pl.ds(g * 16, 16)] = I[b][pl.ds(g * 16, 16)]
            conv(HB[b], F[b], _CH)
            if wait_prev:
                wait_scat(1 - b)
            pltpu.async_copy(F[b], acc.at[J[b]], SS[b], add=True)

        load(0, 0)
        load(1, 1)
        proc(0, wait_prev=False)
        load(2, 0)

        np_steady = (cf - 3) // 2

        def pair(t, carry):
            proc(1)
            load(2 * t + 3, 1)
            proc(0)
            load(2 * t + 4, 0)
            return carry

        lax.fori_loop(0, np_steady, pair, 0)

        loaded = 2 * np_steady + 2
        for k in range(2 * np_steady + 1, cf):
            proc(k % 2)
            nxt = k + 2
            if nxt < cf and nxt > loaded:
                load(nxt, nxt % 2)
                loaded = nxt
        wait_scat((cf - 1) % 2)

        if tail:
            toff = cf * _CH
            pltpu.sync_copy(h_hbm.at[pl.ds(rbase + toff, tail)],
                            hb0.at[pl.ds(0, tail)])
            pltpu.sync_copy(idx_hbm.at[pl.ds(ibase_lo + toff, tail)],
                            it_.at[pl.ds(0, tail)])
            pltpu.sync_copy(idx_hbm.at[pl.ds(ibase_hi + toff, tail)],
                            it_.at[pl.ds(tail, tail)])
            conv(hb0, f0, tail)
            pltpu.sync_copy(f0.at[pl.ds(0, 2 * tail)], acc.at[it_], add=True)

        plsc.subcore_barrier()
        pltpu.sync_copy(acc.at[pl.ds(s * _RPW, _RPW)],
                        out_hbm.at[c, pl.ds(s * _RPW, _RPW)])

      pl.run_scoped(
          _body,
          pltpu.VMEM((_CH, D), jnp.int32),
          pltpu.VMEM((_CH, D), jnp.int32),
          pltpu.VMEM((2 * _CH, D), jnp.float32),
          pltpu.VMEM((2 * _CH, D), jnp.float32),
          pltpu.VMEM((2 * _CH,), jnp.int32),
          pltpu.VMEM((2 * _CH,), jnp.int32),
          pltpu.VMEM((2 * _CH,), jnp.int32),
          pltpu.VMEM((2 * _CH,), jnp.int32),
          pltpu.VMEM((2 * max(tail, 8),), jnp.int32),
      )

    kw = dict(out_type=jax.ShapeDtypeStruct((_NC, N_PAD, D), jnp.float32),
              mesh=_sc_mesh, scratch_types=scratch)
    if chained:
        @functools.partial(pl.kernel, **kw)
        def _scatter_stage(h_hbm, idx_hbm, init_hbm, out_hbm, acc,
                           sw0, sw1, sil0, sih0, sil1, sih1, ss0, ss1):
            _impl(h_hbm, idx_hbm, init_hbm, out_hbm, acc,
                  sw0, sw1, sil0, sih0, sil1, sih1, ss0, ss1)
    else:
        @functools.partial(pl.kernel, **kw)
        def _scatter_stage(h_hbm, idx_hbm, out_hbm, acc,
                           sw0, sw1, sil0, sih0, sil1, sih1, ss0, ss1):
            _impl(h_hbm, idx_hbm, None, out_hbm, acc,
                  sw0, sw1, sil0, sih0, sil1, sih1, ss0, ss1)

    return _scatter_stage


_scatter_stages = [_make_scatter(sl, sl > 0) for sl in range(_NSLICE)]

_NODE_BLK = 2528


def _mlp_body(p_ref, w1_ref, b1_ref, w2_ref, b2_ref, w3_ref, b3_ref, wo_ref,
              out_ref):
    h = p_ref[0] + p_ref[1]
    h = jax.nn.silu(jnp.dot(h, w1_ref[...],
                            preferred_element_type=jnp.float32) + b1_ref[...])
    h = jax.nn.silu(jnp.dot(h, w2_ref[...],
                            preferred_element_type=jnp.float32) + b2_ref[...])
    h = jax.nn.silu(jnp.dot(h, w3_ref[...],
                            preferred_element_type=jnp.float32) + b3_ref[...])
    out_ref[...] = jnp.dot(h, wo_ref[...], preferred_element_type=jnp.float32)


def _mlp_stage(partials, W1, b1, W2, b2, W3, b3, W_out):
    grid = (N_PAD // _NODE_BLK,)
    full = lambda i: (0, 0)
    return pl.pallas_call(
        _mlp_body,
        grid=grid,
        in_specs=[
            pl.BlockSpec((_NC, _NODE_BLK, D), lambda i: (0, i, 0)),
            pl.BlockSpec((D, D), full),
            pl.BlockSpec((1, D), full),
            pl.BlockSpec((D, D), full),
            pl.BlockSpec((1, D), full),
            pl.BlockSpec((D, D), full),
            pl.BlockSpec((1, D), full),
            pl.BlockSpec((D, T), full),
        ],
        out_specs=pl.BlockSpec((_NODE_BLK, T), lambda i: (i, 0)),
        out_shape=jax.ShapeDtypeStruct((N, T), jnp.float32),
    )(partials, W1, b1, W2, b2, W3, b3, W_out)


def kernel(x, rbf, idnb_i, n_atoms, W_rbf, W1, b1, W2, b2, W3, b3, W_out):
    del n_atoms
    p = None
    for sl in range(_NSLICE):
        h_s = _edge_stage(x, rbf, W_rbf, sl)
        args = (h_s, idnb_i) if p is None else (h_s, idnb_i, p)
        p = _scatter_stages[sl](*args)
    return _mlp_stage(p, W1, b1.reshape(1, D), W2, b2.reshape(1, D),
                      W3, b3.reshape(1, D), W_out)
